# initial kernel scaffold (unmeasured)
import jax
import jax.numpy as jnp
from jax import lax
from jax.experimental import pallas as pl
from jax.experimental.pallas import tpu as pltpu

N_DEV = 32
M_PER = 128


def kernel(x, w_mat, scale_x, scale_w):
    m_glob, k_per = x.shape
    k_per_w, n = w_mat.shape
    assert k_per == M_PER and k_per_w == M_PER and m_glob == N_DEV * M_PER

    def body(x_ref, w_ref, sx_ref, sw_ref, out_ref,
             x_blk, w_all, x_send_sem, x_recv_sem, w_send_sem, w_recv_sem):
        me = lax.axis_index("i")
        right = (me + 1) % N_DEV

        def dot(a, b):
            return lax.dot_general(
                a, b, (((1,), (0,)), ((), ())),
                preferred_element_type=jnp.float32,
            )

        w_all[me] = w_ref[...]
        x_blk[me] = x_ref[pl.ds(me * M_PER, M_PER), :]

        for j in range(N_DEV):
            @pl.when(me != j)
            def _():
                send = pltpu.make_async_remote_copy(
                    src_ref=x_ref.at[pl.ds(j * M_PER, M_PER), :],
                    dst_ref=x_blk.at[me],
                    send_sem=x_send_sem,
                    recv_sem=x_recv_sem,
                    device_id=(j,),
                    device_id_type=pl.DeviceIdType.MESH,
                )
                send.start()

        out_ref[...] = dot(x_blk[me], w_all[me])

        for j in range(N_DEV):
            @pl.when(me != j)
            def _():
                recv = pltpu.make_async_remote_copy(
                    src_ref=x_blk.at[j],
                    dst_ref=x_blk.at[j],
                    send_sem=x_send_sem,
                    recv_sem=x_recv_sem,
                    device_id=(j,),
                    device_id_type=pl.DeviceIdType.MESH,
                )
                recv.wait_recv()

        def hop(h, _):
            s = (me - h) % N_DEV
            rdma = pltpu.make_async_remote_copy(
                src_ref=w_all.at[s],
                dst_ref=w_all.at[s],
                send_sem=w_send_sem,
                recv_sem=w_recv_sem,
                device_id=(right,),
                device_id_type=pl.DeviceIdType.MESH,
            )
            rdma.start()

            @pl.when(h > 0)
            def _():
                out_ref[...] += dot(x_blk[s], w_all[s])

            rdma.wait()
            return 0

        lax.fori_loop(0, N_DEV - 1, hop, 0)

        last = (me + 1) % N_DEV
        scale = sx_ref[0] * sw_ref[0]
        out_ref[...] = (out_ref[...] + dot(x_blk[last], w_all[last])) * scale

        for j in range(N_DEV):
            @pl.when(me != j)
            def _():
                sent = pltpu.make_async_remote_copy(
                    src_ref=x_ref.at[pl.ds(j * M_PER, M_PER), :],
                    dst_ref=x_blk.at[me],
                    send_sem=x_send_sem,
                    recv_sem=x_recv_sem,
                    device_id=(j,),
                    device_id_type=pl.DeviceIdType.MESH,
                )
                sent.wait_send()

    return pl.pallas_call(
        body,
        out_shape=jax.ShapeDtypeStruct((M_PER, n), jnp.float32),
        in_specs=[
            pl.BlockSpec(memory_space=pltpu.VMEM),
            pl.BlockSpec(memory_space=pltpu.VMEM),
            pl.BlockSpec(memory_space=pltpu.SMEM),
            pl.BlockSpec(memory_space=pltpu.SMEM),
        ],
        out_specs=pl.BlockSpec(memory_space=pltpu.VMEM),
        scratch_shapes=[
            pltpu.VMEM((N_DEV, M_PER, M_PER), x.dtype),
            pltpu.VMEM((N_DEV, M_PER, n), w_mat.dtype),
            pltpu.SemaphoreType.DMA,
            pltpu.SemaphoreType.DMA,
            pltpu.SemaphoreType.DMA,
            pltpu.SemaphoreType.DMA,
        ],
        compiler_params=pltpu.CompilerParams(collective_id=0),
    )(x, w_mat, scale_x, scale_w)


# baseline (device time: 437302 ns/iter reference)
import jax
import jax.numpy as jnp
from jax import lax
from jax.experimental import pallas as pl
from jax.experimental.pallas import tpu as pltpu

N_DEV = 32
M_PER = 128


def kernel(x, w_mat, scale_x, scale_w):
    m_glob, k_per = x.shape
    k_per_w, n = w_mat.shape
    assert k_per == M_PER and k_per_w == M_PER and m_glob == N_DEV * M_PER

    def body(x_ref, w_ref, sx_ref, sw_ref, out_ref,
             x8, x_blk, w_all, x_send_sem, x_recv_sem, w_send_sem, w_recv_sem):
        me = lax.axis_index("i")
        right = (me + 1) % N_DEV

        def dot(a, b):
            return lax.dot_general(
                a, b, (((1,), (0,)), ((), ())),
                preferred_element_type=jnp.float32,
            )

        x8[...] = x_ref[...].astype(jnp.float8_e4m3fn)
        w_all[me] = w_ref[...].astype(jnp.float8_e4m3fn)
        x_blk[me] = x8[pl.ds(me * M_PER, M_PER), :]

        for j in range(N_DEV):
            @pl.when(me != j)
            def _():
                send = pltpu.make_async_remote_copy(
                    src_ref=x8.at[pl.ds(j * M_PER, M_PER), :],
                    dst_ref=x_blk.at[me],
                    send_sem=x_send_sem,
                    recv_sem=x_recv_sem,
                    device_id=(j,),
                    device_id_type=pl.DeviceIdType.MESH,
                )
                send.start()

        out_ref[...] = dot(x_blk[me], w_all[me])

        for j in range(N_DEV):
            @pl.when(me != j)
            def _():
                recv = pltpu.make_async_remote_copy(
                    src_ref=x_blk.at[j],
                    dst_ref=x_blk.at[j],
                    send_sem=x_send_sem,
                    recv_sem=x_recv_sem,
                    device_id=(j,),
                    device_id_type=pl.DeviceIdType.MESH,
                )
                recv.wait_recv()

        def hop(h, _):
            s = (me - h) % N_DEV
            rdma = pltpu.make_async_remote_copy(
                src_ref=w_all.at[s],
                dst_ref=w_all.at[s],
                send_sem=w_send_sem,
                recv_sem=w_recv_sem,
                device_id=(right,),
                device_id_type=pl.DeviceIdType.MESH,
            )
            rdma.start()

            @pl.when(h > 0)
            def _():
                out_ref[...] += dot(x_blk[s], w_all[s])

            rdma.wait()
            return 0

        lax.fori_loop(0, N_DEV - 1, hop, 0)

        last = (me + 1) % N_DEV
        scale = sx_ref[0] * sw_ref[0]
        out_ref[...] = (out_ref[...] + dot(x_blk[last], w_all[last])) * scale

        for j in range(N_DEV):
            @pl.when(me != j)
            def _():
                sent = pltpu.make_async_remote_copy(
                    src_ref=x8.at[pl.ds(j * M_PER, M_PER), :],
                    dst_ref=x_blk.at[me],
                    send_sem=x_send_sem,
                    recv_sem=x_recv_sem,
                    device_id=(j,),
                    device_id_type=pl.DeviceIdType.MESH,
                )
                sent.wait_send()

    return pl.pallas_call(
        body,
        out_shape=jax.ShapeDtypeStruct((M_PER, n), jnp.float32),
        in_specs=[
            pl.BlockSpec(memory_space=pltpu.VMEM),
            pl.BlockSpec(memory_space=pltpu.VMEM),
            pl.BlockSpec(memory_space=pltpu.SMEM),
            pl.BlockSpec(memory_space=pltpu.SMEM),
        ],
        out_specs=pl.BlockSpec(memory_space=pltpu.VMEM),
        scratch_shapes=[
            pltpu.VMEM((m_glob, k_per), jnp.float8_e4m3fn),
            pltpu.VMEM((N_DEV, M_PER, M_PER), jnp.float8_e4m3fn),
            pltpu.VMEM((N_DEV, M_PER, n), jnp.float8_e4m3fn),
            pltpu.SemaphoreType.DMA,
            pltpu.SemaphoreType.DMA,
            pltpu.SemaphoreType.DMA,
            pltpu.SemaphoreType.DMA,
        ],
        compiler_params=pltpu.CompilerParams(
            vmem_limit_bytes=50 * 1024 * 1024,
        ),
    )(x, w_mat, scale_x, scale_w)


# device time: 430110 ns/iter; 1.0167x vs baseline; 1.0167x over previous
import jax
import jax.numpy as jnp
from jax import lax
from jax.experimental import pallas as pl
from jax.experimental.pallas import tpu as pltpu

N_DEV = 32
M_PER = 128


def kernel(x, w_mat, scale_x, scale_w):
    m_glob, k_per = x.shape
    k_per_w, n = w_mat.shape
    assert k_per == M_PER and k_per_w == M_PER and m_glob == N_DEV * M_PER
    nh = n // 2

    def body(x_ref, w_ref, sx_ref, sw_ref, out_ref,
             x8, x_blk, w_cw, w_ccw,
             x_send_sem, x_recv_sem,
             cw_send_sem, cw_recv_sem, ccw_send_sem, ccw_recv_sem):
        me = lax.axis_index("i")
        right = (me + 1) % N_DEV
        left = (me + N_DEV - 1) % N_DEV

        def dot(a, b):
            return lax.dot_general(
                a, b, (((1,), (0,)), ((), ())),
                preferred_element_type=jnp.float32,
            )

        x8[...] = x_ref[...].astype(jnp.float8_e4m3fn)
        w_cw[me] = w_ref[:, :nh].astype(jnp.float8_e4m3fn)
        w_ccw[me] = w_ref[:, nh:].astype(jnp.float8_e4m3fn)
        x_blk[me] = x8[pl.ds(me * M_PER, M_PER), :]

        for j in range(N_DEV):
            @pl.when(me != j)
            def _():
                send = pltpu.make_async_remote_copy(
                    src_ref=x8.at[pl.ds(j * M_PER, M_PER), :],
                    dst_ref=x_blk.at[me],
                    send_sem=x_send_sem,
                    recv_sem=x_recv_sem,
                    device_id=(j,),
                    device_id_type=pl.DeviceIdType.MESH,
                )
                send.start()

        out_ref[:, :nh] = dot(x_blk[me], w_cw[me])
        out_ref[:, nh:] = dot(x_blk[me], w_ccw[me])

        for j in range(N_DEV):
            @pl.when(me != j)
            def _():
                recv = pltpu.make_async_remote_copy(
                    src_ref=x_blk.at[j],
                    dst_ref=x_blk.at[j],
                    send_sem=x_send_sem,
                    recv_sem=x_recv_sem,
                    device_id=(j,),
                    device_id_type=pl.DeviceIdType.MESH,
                )
                recv.wait_recv()

        def hop(h, _):
            s_cw = (me + N_DEV - h) % N_DEV
            s_ccw = (me + h) % N_DEV
            rdma_cw = pltpu.make_async_remote_copy(
                src_ref=w_cw.at[s_cw],
                dst_ref=w_cw.at[s_cw],
                send_sem=cw_send_sem,
                recv_sem=cw_recv_sem,
                device_id=(right,),
                device_id_type=pl.DeviceIdType.MESH,
            )
            rdma_ccw = pltpu.make_async_remote_copy(
                src_ref=w_ccw.at[s_ccw],
                dst_ref=w_ccw.at[s_ccw],
                send_sem=ccw_send_sem,
                recv_sem=ccw_recv_sem,
                device_id=(left,),
                device_id_type=pl.DeviceIdType.MESH,
            )
            rdma_cw.start()
            rdma_ccw.start()

            @pl.when(h > 0)
            def _():
                out_ref[:, :nh] += dot(x_blk[s_cw], w_cw[s_cw])
                out_ref[:, nh:] += dot(x_blk[s_ccw], w_ccw[s_ccw])

            rdma_cw.wait()
            rdma_ccw.wait()
            return 0

        lax.fori_loop(0, N_DEV - 1, hop, 0)

        last_cw = right
        last_ccw = left
        scale = sx_ref[0] * sw_ref[0]
        out_ref[:, :nh] = (
            out_ref[:, :nh] + dot(x_blk[last_cw], w_cw[last_cw])
        ) * scale
        out_ref[:, nh:] = (
            out_ref[:, nh:] + dot(x_blk[last_ccw], w_ccw[last_ccw])
        ) * scale

        for j in range(N_DEV):
            @pl.when(me != j)
            def _():
                sent = pltpu.make_async_remote_copy(
                    src_ref=x8.at[pl.ds(j * M_PER, M_PER), :],
                    dst_ref=x_blk.at[me],
                    send_sem=x_send_sem,
                    recv_sem=x_recv_sem,
                    device_id=(j,),
                    device_id_type=pl.DeviceIdType.MESH,
                )
                sent.wait_send()

    return pl.pallas_call(
        body,
        out_shape=jax.ShapeDtypeStruct((M_PER, n), jnp.float32),
        in_specs=[
            pl.BlockSpec(memory_space=pltpu.VMEM),
            pl.BlockSpec(memory_space=pltpu.VMEM),
            pl.BlockSpec(memory_space=pltpu.SMEM),
            pl.BlockSpec(memory_space=pltpu.SMEM),
        ],
        out_specs=pl.BlockSpec(memory_space=pltpu.VMEM),
        scratch_shapes=[
            pltpu.VMEM((m_glob, k_per), jnp.float8_e4m3fn),
            pltpu.VMEM((N_DEV, M_PER, M_PER), jnp.float8_e4m3fn),
            pltpu.VMEM((N_DEV, M_PER, nh), jnp.float8_e4m3fn),
            pltpu.VMEM((N_DEV, M_PER, nh), jnp.float8_e4m3fn),
            pltpu.SemaphoreType.DMA,
            pltpu.SemaphoreType.DMA,
            pltpu.SemaphoreType.DMA,
            pltpu.SemaphoreType.DMA,
            pltpu.SemaphoreType.DMA,
            pltpu.SemaphoreType.DMA,
        ],
        compiler_params=pltpu.CompilerParams(
            vmem_limit_bytes=50 * 1024 * 1024,
        ),
    )(x, w_mat, scale_x, scale_w)


# device time: 259233 ns/iter; 1.6869x vs baseline; 1.6592x over previous
import jax
import jax.numpy as jnp
from jax import lax
from jax.experimental import pallas as pl
from jax.experimental.pallas import tpu as pltpu

N_DEV = 32
M_PER = 128

_RING = [0, 8, 16, 24, 27, 19, 11, 3, 4, 12, 20, 28, 31, 23, 15, 7,
         6, 14, 22, 30, 29, 21, 13, 5, 2, 10, 18, 26, 25, 17, 9, 1]
_POS = [0] * N_DEV
for _p, _i in enumerate(_RING):
    _POS[_i] = _p


def kernel(x, w_mat, scale_x, scale_w):
    m_glob, k_per = x.shape
    k_per_w, n = w_mat.shape
    assert k_per == M_PER and k_per_w == M_PER and m_glob == N_DEV * M_PER
    nh = n // 2

    def body(x_ref, w_ref, sx_ref, sw_ref, out_ref,
             x8, x_blk, w_cw, w_ccw,
             x_send_sem, x_recv_sem,
             cw_send_sem, cw_recv_sem, ccw_send_sem, ccw_recv_sem):
        me = lax.axis_index("i")

        iota = lax.broadcasted_iota(jnp.int32, (1, N_DEV), 1)
        zc = iota // 8
        wc = iota % 8
        yc = wc // 2
        xc = (wc + yc) % 2
        zig = jnp.where(yc % 2 == 0, zc, 3 - zc)
        r1 = 3 - yc
        zag = jnp.where(r1 % 2 == 0, zc, 3 - zc)
        pos_c = jnp.where(xc == 0, 4 * yc + zig, 16 + 4 * r1 + zag)

        def lookup(table, idx):
            return jnp.sum(jnp.where(iota == idx, table, 0))

        p = lookup(pos_c, me)
        rp = (p + 1) % N_DEV
        lp = (p + N_DEV - 1) % N_DEV
        right = jnp.sum(jnp.where(pos_c == rp, iota, 0))
        left = jnp.sum(jnp.where(pos_c == lp, iota, 0))

        def dot(a, b):
            return lax.dot_general(
                a, b, (((1,), (0,)), ((), ())),
                preferred_element_type=jnp.float32,
            )

        x8[...] = x_ref[...].astype(jnp.float8_e4m3fn)
        w_cw[p] = w_ref[:, :nh].astype(jnp.float8_e4m3fn)
        w_ccw[p] = w_ref[:, nh:].astype(jnp.float8_e4m3fn)
        x_blk[p] = x8[pl.ds(me * M_PER, M_PER), :]

        for j in range(N_DEV):
            @pl.when(me != j)
            def _():
                send = pltpu.make_async_remote_copy(
                    src_ref=x8.at[pl.ds(j * M_PER, M_PER), :],
                    dst_ref=x_blk.at[p],
                    send_sem=x_send_sem,
                    recv_sem=x_recv_sem,
                    device_id=(j,),
                    device_id_type=pl.DeviceIdType.MESH,
                )
                send.start()

        out_ref[:, :nh] = dot(x_blk[p], w_cw[p])
        out_ref[:, nh:] = dot(x_blk[p], w_ccw[p])

        for j in range(N_DEV):
            @pl.when(me != j)
            def _():
                recv = pltpu.make_async_remote_copy(
                    src_ref=x_blk.at[j],
                    dst_ref=x_blk.at[j],
                    send_sem=x_send_sem,
                    recv_sem=x_recv_sem,
                    device_id=(j,),
                    device_id_type=pl.DeviceIdType.MESH,
                )
                recv.wait_recv()

        def hop(h, _):
            s_cw = (p + N_DEV - h) % N_DEV
            s_ccw = (p + h) % N_DEV
            rdma_cw = pltpu.make_async_remote_copy(
                src_ref=w_cw.at[s_cw],
                dst_ref=w_cw.at[s_cw],
                send_sem=cw_send_sem,
                recv_sem=cw_recv_sem,
                device_id=(right,),
                device_id_type=pl.DeviceIdType.MESH,
            )
            rdma_ccw = pltpu.make_async_remote_copy(
                src_ref=w_ccw.at[s_ccw],
                dst_ref=w_ccw.at[s_ccw],
                send_sem=ccw_send_sem,
                recv_sem=ccw_recv_sem,
                device_id=(left,),
                device_id_type=pl.DeviceIdType.MESH,
            )
            rdma_cw.start()
            rdma_ccw.start()

            @pl.when(h > 0)
            def _():
                out_ref[:, :nh] += dot(x_blk[s_cw], w_cw[s_cw])
                out_ref[:, nh:] += dot(x_blk[s_ccw], w_ccw[s_ccw])

            rdma_cw.wait()
            rdma_ccw.wait()
            return 0

        lax.fori_loop(0, N_DEV - 1, hop, 0)

        last_cw = rp
        last_ccw = lp
        scale = sx_ref[0] * sw_ref[0]
        out_ref[:, :nh] = (
            out_ref[:, :nh] + dot(x_blk[last_cw], w_cw[last_cw])
        ) * scale
        out_ref[:, nh:] = (
            out_ref[:, nh:] + dot(x_blk[last_ccw], w_ccw[last_ccw])
        ) * scale

        for j in range(N_DEV):
            @pl.when(me != j)
            def _():
                sent = pltpu.make_async_remote_copy(
                    src_ref=x8.at[pl.ds(j * M_PER, M_PER), :],
                    dst_ref=x_blk.at[p],
                    send_sem=x_send_sem,
                    recv_sem=x_recv_sem,
                    device_id=(j,),
                    device_id_type=pl.DeviceIdType.MESH,
                )
                sent.wait_send()

    return pl.pallas_call(
        body,
        out_shape=jax.ShapeDtypeStruct((M_PER, n), jnp.float32),
        in_specs=[
            pl.BlockSpec(memory_space=pltpu.VMEM),
            pl.BlockSpec(memory_space=pltpu.VMEM),
            pl.BlockSpec(memory_space=pltpu.SMEM),
            pl.BlockSpec(memory_space=pltpu.SMEM),
        ],
        out_specs=pl.BlockSpec(memory_space=pltpu.VMEM),
        scratch_shapes=[
            pltpu.VMEM((m_glob, k_per), jnp.float8_e4m3fn),
            pltpu.VMEM((N_DEV, M_PER, M_PER), jnp.float8_e4m3fn),
            pltpu.VMEM((N_DEV, M_PER, nh), jnp.float8_e4m3fn),
            pltpu.VMEM((N_DEV, M_PER, nh), jnp.float8_e4m3fn),
            pltpu.SemaphoreType.DMA,
            pltpu.SemaphoreType.DMA,
            pltpu.SemaphoreType.DMA,
            pltpu.SemaphoreType.DMA,
            pltpu.SemaphoreType.DMA,
            pltpu.SemaphoreType.DMA,
        ],
        compiler_params=pltpu.CompilerParams(
            vmem_limit_bytes=50 * 1024 * 1024,
        ),
    )(x, w_mat, scale_x, scale_w)


# device time: 203669 ns/iter; 2.1471x vs baseline; 1.2728x over previous
import jax
import jax.numpy as jnp
from jax import lax
from jax.experimental import pallas as pl
from jax.experimental.pallas import tpu as pltpu

N_DEV = 32
M_PER = 128

_RING = [0, 8, 16, 24, 27, 19, 11, 3, 4, 12, 20, 28, 31, 23, 15, 7,
         6, 14, 22, 30, 29, 21, 13, 5, 2, 10, 18, 26, 25, 17, 9, 1]


def kernel(x, w_mat, scale_x, scale_w):
    m_glob, k_per = x.shape
    k_per_w, n = w_mat.shape
    assert k_per == M_PER and k_per_w == M_PER and m_glob == N_DEV * M_PER
    nq = n // 4

    def body(x_ref, w_ref, sx_ref, sw_ref, out_ref,
             x8, x_blk, w_cw0, w_cw1, w_ccw0, w_ccw1,
             x_send_sem, x_recv_sem,
             cw_s0, cw_r0, cw_s1, cw_r1,
             ccw_s0, ccw_r0, ccw_s1, ccw_r1):
        me = lax.axis_index("i")

        iota = lax.broadcasted_iota(jnp.int32, (1, N_DEV), 1)
        zc = iota // 8
        wc = iota % 8
        yc = wc // 2
        xc = (wc + yc) % 2
        zig = jnp.where(yc % 2 == 0, zc, 3 - zc)
        r1 = 3 - yc
        zag = jnp.where(r1 % 2 == 0, zc, 3 - zc)
        pos_c = jnp.where(xc == 0, 4 * yc + zig, 16 + 4 * r1 + zag)

        p = jnp.sum(jnp.where(iota == me, pos_c, 0))
        rp = (p + 1) % N_DEV
        lp = (p + N_DEV - 1) % N_DEV
        right = jnp.sum(jnp.where(pos_c == rp, iota, 0))
        left = jnp.sum(jnp.where(pos_c == lp, iota, 0))

        def dot(a, b):
            return lax.dot_general(
                a, b, (((1,), (0,)), ((), ())),
                preferred_element_type=jnp.float32,
            )

        lanes = (w_cw0, w_cw1, w_ccw0, w_ccw1)
        sems = ((cw_s0, cw_r0), (cw_s1, cw_r1),
                (ccw_s0, ccw_r0), (ccw_s1, ccw_r1))
        targets = (right, right, left, left)

        def lane_rdma(lane, q):
            buf = lanes[lane]
            s_sem, r_sem = sems[lane]
            return pltpu.make_async_remote_copy(
                src_ref=buf.at[q],
                dst_ref=buf.at[q],
                send_sem=s_sem,
                recv_sem=r_sem,
                device_id=(targets[lane],),
                device_id_type=pl.DeviceIdType.MESH,
            )

        x8[...] = x_ref[...].astype(jnp.float8_e4m3fn)
        for lane in range(4):
            lanes[lane][p] = w_ref[:, lane * nq:(lane + 1) * nq].astype(
                jnp.float8_e4m3fn)
        x_blk[p] = x8[pl.ds(me * M_PER, M_PER), :]

        for j in range(N_DEV):
            @pl.when(me != j)
            def _():
                send = pltpu.make_async_remote_copy(
                    src_ref=x8.at[pl.ds(j * M_PER, M_PER), :],
                    dst_ref=x_blk.at[p],
                    send_sem=x_send_sem,
                    recv_sem=x_recv_sem,
                    device_id=(j,),
                    device_id_type=pl.DeviceIdType.MESH,
                )
                send.start()

        for lane in range(4):
            lane_rdma(lane, p).start()

        for lane in range(4):
            out_ref[:, lane * nq:(lane + 1) * nq] = dot(x_blk[p], lanes[lane][p])

        for j in range(N_DEV):
            @pl.when(me != j)
            def _():
                recv = pltpu.make_async_remote_copy(
                    src_ref=x_blk.at[j],
                    dst_ref=x_blk.at[j],
                    send_sem=x_send_sem,
                    recv_sem=x_recv_sem,
                    device_id=(j,),
                    device_id_type=pl.DeviceIdType.MESH,
                )
                recv.wait_recv()

        def hop(h, _):
            q_cw = (p + N_DEV - h) % N_DEV
            q_ccw = (p + h) % N_DEV
            for lane in range(4):
                q = q_cw if lane < 2 else q_ccw
                d = lane_rdma(lane, q)
                d.wait_recv()
                d.wait_send()
                d.start()
            for lane in range(4):
                q = q_cw if lane < 2 else q_ccw
                out_ref[:, lane * nq:(lane + 1) * nq] += dot(
                    x_blk[q], lanes[lane][q])
            return 0

        lax.fori_loop(1, N_DEV - 1, hop, 0)

        scale = sx_ref[0] * sw_ref[0]
        for lane in range(4):
            q = rp if lane < 2 else lp
            d = lane_rdma(lane, q)
            d.wait_recv()
            d.wait_send()
            out_ref[:, lane * nq:(lane + 1) * nq] = (
                out_ref[:, lane * nq:(lane + 1) * nq]
                + dot(x_blk[q], lanes[lane][q])
            ) * scale

        for j in range(N_DEV):
            @pl.when(me != j)
            def _():
                sent = pltpu.make_async_remote_copy(
                    src_ref=x8.at[pl.ds(j * M_PER, M_PER), :],
                    dst_ref=x_blk.at[p],
                    send_sem=x_send_sem,
                    recv_sem=x_recv_sem,
                    device_id=(j,),
                    device_id_type=pl.DeviceIdType.MESH,
                )
                sent.wait_send()

    return pl.pallas_call(
        body,
        out_shape=jax.ShapeDtypeStruct((M_PER, n), jnp.float32),
        in_specs=[
            pl.BlockSpec(memory_space=pltpu.VMEM),
            pl.BlockSpec(memory_space=pltpu.VMEM),
            pl.BlockSpec(memory_space=pltpu.SMEM),
            pl.BlockSpec(memory_space=pltpu.SMEM),
        ],
        out_specs=pl.BlockSpec(memory_space=pltpu.VMEM),
        scratch_shapes=[
            pltpu.VMEM((m_glob, k_per), jnp.float8_e4m3fn),
            pltpu.VMEM((N_DEV, M_PER, M_PER), jnp.float8_e4m3fn),
            pltpu.VMEM((N_DEV, M_PER, nq), jnp.float8_e4m3fn),
            pltpu.VMEM((N_DEV, M_PER, nq), jnp.float8_e4m3fn),
            pltpu.VMEM((N_DEV, M_PER, nq), jnp.float8_e4m3fn),
            pltpu.VMEM((N_DEV, M_PER, nq), jnp.float8_e4m3fn),
            pltpu.SemaphoreType.DMA,
            pltpu.SemaphoreType.DMA,
            pltpu.SemaphoreType.DMA,
            pltpu.SemaphoreType.DMA,
            pltpu.SemaphoreType.DMA,
            pltpu.SemaphoreType.DMA,
            pltpu.SemaphoreType.DMA,
            pltpu.SemaphoreType.DMA,
            pltpu.SemaphoreType.DMA,
            pltpu.SemaphoreType.DMA,
        ],
        compiler_params=pltpu.CompilerParams(
            vmem_limit_bytes=50 * 1024 * 1024,
        ),
    )(x, w_mat, scale_x, scale_w)
